# baseline (speedup 1.0000x reference)
import functools

import jax
import jax.numpy as jnp
from jax import lax
from jax.experimental import pallas as pl
from jax.experimental.pallas import tpu as pltpu
from jax.experimental.pallas import tpu_sc as plsc

_R_CUTOFF = 10.0
_N = 10000
_E = 320000
_F = 128
_DF = 16

_NC, _NS, _NW = 2, 16, 32
_CH = 128
_DROWS = 320
_SLICES = (64000, 83200, 86400, 86400)


def _silu(v):
    return v * jax.nn.sigmoid(v)


def _node_pre(h, w_a, w_b, b1m):
    bn = 2000

    def body(h_ref, wa_ref, wb_ref, b_ref, g0_ref, g1_ref):
        hv = h_ref[...]
        g0_ref[...] = jnp.dot(hv, wa_ref[...], preferred_element_type=jnp.float32) + b_ref[...]
        g1_ref[...] = jnp.dot(hv, wb_ref[...], preferred_element_type=jnp.float32)

    return pl.pallas_call(
        body,
        grid=(_N // bn,),
        in_specs=[
            pl.BlockSpec((bn, _F), lambda i: (i, 0)),
            pl.BlockSpec((_F, _F), lambda i: (0, 0)),
            pl.BlockSpec((_F, _F), lambda i: (0, 0)),
            pl.BlockSpec((1, _F), lambda i: (0, 0)),
        ],
        out_specs=[
            pl.BlockSpec((bn, _F), lambda i: (i, 0)),
            pl.BlockSpec((bn, _F), lambda i: (i, 0)),
        ],
        out_shape=[
            jax.ShapeDtypeStruct((_N, _F), jnp.float32),
            jax.ShapeDtypeStruct((_N, _F), jnp.float32),
        ],
    )(h, w_a, w_b, b1m.reshape(1, _F))


def _sc_gather(g0, g1, px4, e0, e1, ne):
    mesh = plsc.VectorSubcoreMesh(core_axis_name="c", subcore_axis_name="s")
    nchunk = ne // _CH
    iters = -(-nchunk // _NW)
    pairs = -(-iters // 2)

    @functools.partial(
        pl.kernel,
        out_type=(
            jax.ShapeDtypeStruct((ne, _F), jnp.float32),
            jax.ShapeDtypeStruct((16, ne), jnp.float32),
        ),
        mesh=mesh,
        scratch_types=[
            pltpu.VMEM((_CH,), jnp.int32),
            pltpu.VMEM((_CH,), jnp.int32),
            pltpu.VMEM((_CH,), jnp.int32),
            pltpu.VMEM((_CH,), jnp.int32),
            pltpu.VMEM((_CH, _F), jnp.float32),
            pltpu.VMEM((_CH, _F), jnp.float32),
            pltpu.VMEM((_CH, _F), jnp.float32),
            pltpu.VMEM((_CH, _F), jnp.float32),
            pltpu.VMEM((16, _CH), jnp.float32),
            pltpu.VMEM((16, _CH), jnp.float32),
            pltpu.VMEM((_N * 4,), jnp.float32),
            pltpu.SemaphoreType.DMA,
            pltpu.SemaphoreType.DMA,
        ],
        compiler_params=pltpu.CompilerParams(needs_layout_passes=False),
    )
    def k(g0_h, g1_h, px_h, e0_h, e1_h, pre_h, diff_h,
          idx0a, idx1a, idx0b, idx1b, r0a, r1a, r0b, r1b, p0a, p0b, pxv, semA, semB):
        c = lax.axis_index("c")
        s = lax.axis_index("s")
        wid = s * _NC + c
        pltpu.sync_copy(px_h, pxv)

        def zrow(rr, c2):
            for kk in range(_CH // 16):
                p0a[rr, pl.ds(kk * 16, 16)] = jnp.zeros((16,), jnp.float32)
                p0b[rr, pl.ds(kk * 16, 16)] = jnp.zeros((16,), jnp.float32)
            return c2

        lax.fori_loop(0, 16, zrow, 0)
        iota = lax.iota(jnp.int32, 16)

        def issue(ci, idx0, idx1, r0, r1, sem):
            base = ci * _CH
            pltpu.sync_copy(e0_h.at[pl.ds(base, _CH)], idx0)
            pltpu.sync_copy(e1_h.at[pl.ds(base, _CH)], idx1)
            pltpu.async_copy(g0_h.at[idx0], r0, sem)
            pltpu.async_copy(g1_h.at[idx1], r1, sem)

        def finish(ci, idx0, idx1, r0, r1, p0, sem):
            base = ci * _CH
            pltpu.make_async_copy(g0_h.at[idx0], r0, sem).wait()
            pltpu.make_async_copy(g1_h.at[idx1], r1, sem).wait()
            for kk in range(_CH // 16):
                cols = kk * 16 + iota
                iv0 = idx0[pl.ds(kk * 16, 16)] * 4
                iv1 = idx1[pl.ds(kk * 16, 16)] * 4
                for cc in range(3):
                    a = plsc.load_gather(pxv, [iv0 + cc])
                    b = plsc.load_gather(pxv, [iv1 + cc])
                    plsc.store_scatter(
                        p0, [jnp.full((16,), cc, jnp.int32), cols], a - b)

            def row(rr, c2):
                for kk in range(_F // 16):
                    sl = pl.ds(kk * 16, 16)
                    r0[rr, sl] = r0[rr, sl] + r1[rr, sl]
                return c2

            lax.fori_loop(0, _CH, row, 0)
            pltpu.sync_copy(r0, pre_h.at[pl.ds(base, _CH)])
            pltpu.sync_copy(p0, diff_h.at[pl.ds(0, 16), pl.ds(base, _CH)])

        @pl.when(wid < nchunk)
        def _():
            issue(wid, idx0a, idx1a, r0a, r1a, semA)

        def body(t, carry):
            ca = wid + (2 * t) * _NW
            cb = wid + (2 * t + 1) * _NW
            ca2 = wid + (2 * t + 2) * _NW

            @pl.when(cb < nchunk)
            def _():
                issue(cb, idx0b, idx1b, r0b, r1b, semB)

            @pl.when(ca < nchunk)
            def _():
                finish(ca, idx0a, idx1a, r0a, r1a, p0a, semA)

            @pl.when(ca2 < nchunk)
            def _():
                issue(ca2, idx0a, idx1a, r0a, r1a, semA)

            @pl.when(cb < nchunk)
            def _():
                finish(cb, idx0b, idx1b, r0b, r1b, p0b, semB)

            return carry

        lax.fori_loop(0, pairs, body, 0)

    return k(g0, g1, px4, e0, e1)


def _edge_mlp(pre, dvt, w_emb, means2, coef, W2m, b2m, Wa, ba, W1x, b1x, W2x, ne):
    be = 3200

    def body(pre_ref, dvt_ref, wemb_ref, means_ref, coef_ref, w2m_ref, b2m_ref,
             wa_ref, ba_ref, w1x_ref, b1x_ref, w2x_ref, mm_ref, dispt_ref):
        dvt = dvt_ref[...]
        dvt2 = dvt * dvt
        onescol = jnp.ones((16, 1), jnp.float32)
        d2c = lax.dot_general(dvt2, onescol, (((0,), (0,)), ((), ())))
        dist = jnp.sqrt(d2c)
        mask = (dist <= _R_CUTOFF).astype(jnp.float32)
        emb = jnp.exp(coef_ref[...] * (dist - means_ref[...]) ** 2)
        preact = pre_ref[...] + jnp.dot(emb, wemb_ref[...], preferred_element_type=jnp.float32)
        m1 = _silu(preact)
        m2 = _silu(jnp.dot(m1, w2m_ref[...], preferred_element_type=jnp.float32) + b2m_ref[...])
        att = jax.nn.sigmoid(jnp.dot(m2, wa_ref[...], preferred_element_type=jnp.float32) + ba_ref[...])
        m = m2 * att
        mm_ref[...] = m * mask
        xh = _silu(jnp.dot(m, w1x_ref[...], preferred_element_type=jnp.float32) + b1x_ref[...])
        dmt = jnp.tanh(lax.dot_general(w2x_ref[...], xh, (((0,), (1,)), ((), ()))))
        d2r = lax.dot_general(onescol, dvt2, (((0,), (0,)), ((), ())))
        distr = jnp.sqrt(d2r)
        maskr = (distr <= _R_CUTOFF).astype(jnp.float32)
        dispt_ref[...] = dvt * (dmt * maskr / distr)

    return pl.pallas_call(
        body,
        grid=(ne // be,),
        in_specs=[
            pl.BlockSpec((be, _F), lambda i: (i, 0)),
            pl.BlockSpec((16, be), lambda i: (0, i)),
            pl.BlockSpec((_DF, _F), lambda i: (0, 0)),
            pl.BlockSpec((1, _DF), lambda i: (0, 0)),
            pl.BlockSpec((1, _DF), lambda i: (0, 0)),
            pl.BlockSpec((_F, _F), lambda i: (0, 0)),
            pl.BlockSpec((1, _F), lambda i: (0, 0)),
            pl.BlockSpec((_F, 1), lambda i: (0, 0)),
            pl.BlockSpec((1, 1), lambda i: (0, 0)),
            pl.BlockSpec((_F, _F), lambda i: (0, 0)),
            pl.BlockSpec((1, _F), lambda i: (0, 0)),
            pl.BlockSpec((_F, 1), lambda i: (0, 0)),
        ],
        out_specs=[
            pl.BlockSpec((be, _F), lambda i: (i, 0)),
            pl.BlockSpec((16, be), lambda i: (0, i)),
        ],
        out_shape=[
            jax.ShapeDtypeStruct((ne, _F), jnp.float32),
            jax.ShapeDtypeStruct((16, ne), jnp.float32),
        ],
    )(pre, dvt, w_emb, means2, coef, W2m, b2m.reshape(1, _F), Wa, ba.reshape(1, 1),
      W1x, b1x.reshape(1, _F), W2x)


def _sc_scatter_msg(mm, e0, ne):
    mesh = plsc.VectorSubcoreMesh(core_axis_name="c", subcore_axis_name="s")
    nchunk = ne // _CH
    iters = -(-nchunk // _NW)

    @functools.partial(
        pl.kernel,
        out_type=jax.ShapeDtypeStruct((_NC, _N, _F), jnp.float32),
        mesh=mesh,
        scratch_types=[
            pltpu.VMEM((_CH,), jnp.int32),
            pltpu.VMEM((_CH,), jnp.int32),
            pltpu.VMEM((_CH, _F), jnp.float32),
            pltpu.VMEM((_CH, _F), jnp.float32),
            pltpu.VMEM_SHARED((_N, _F), jnp.float32),
            pltpu.SemaphoreType.DMA,
            pltpu.SemaphoreType.DMA,
        ],
        compiler_params=pltpu.CompilerParams(needs_layout_passes=False),
    )
    def k(mm_h, e0_h, omsg_h, idx0a, idx0b, mrowa, mrowb, macc, semA, semB):
        c = lax.axis_index("c")
        s = lax.axis_index("s")
        wid = s * _NC + c
        pairs = -(-iters // 2)

        def zrow(rr, c2):
            for kk in range(_F // 16):
                mrowa[rr, pl.ds(kk * 16, 16)] = jnp.zeros((16,), jnp.float32)
            return c2

        lax.fori_loop(0, _CH, zrow, 0)
        base_r = s * 624
        for t in range(5):
            pltpu.sync_copy(mrowa, macc.at[pl.ds(base_r + t * _CH, _CH)])
        plsc.subcore_barrier()

        def fire(ci, idx0, mrow, sem):
            base = ci * _CH
            pltpu.sync_copy(e0_h.at[pl.ds(base, _CH)], idx0)
            pltpu.sync_copy(mm_h.at[pl.ds(base, _CH)], mrow)
            pltpu.async_copy(mrow, macc.at[idx0], sem, add=True)

        def drain(idx0, mrow, sem):
            pltpu.make_async_copy(mrow, macc.at[idx0], sem).wait()

        @pl.when(wid < nchunk)
        def _():
            fire(wid, idx0a, mrowa, semA)

        def body(t, carry):
            cb = wid + (2 * t + 1) * _NW
            cb_prev = wid + (2 * t - 1) * _NW
            ca = wid + (2 * t) * _NW
            ca2 = wid + (2 * t + 2) * _NW

            @pl.when(jnp.logical_and(t > 0, cb_prev < nchunk))
            def _():
                drain(idx0b, mrowb, semB)

            @pl.when(cb < nchunk)
            def _():
                fire(cb, idx0b, mrowb, semB)

            @pl.when(ca < nchunk)
            def _():
                drain(idx0a, mrowa, semA)

            @pl.when(ca2 < nchunk)
            def _():
                fire(ca2, idx0a, mrowa, semA)

            return carry

        lax.fori_loop(0, pairs, body, 0)

        @pl.when(wid + (2 * pairs - 1) * _NW < nchunk)
        def _():
            drain(idx0b, mrowb, semB)

        plsc.subcore_barrier()
        sl = pl.ds(base_r, 624)
        pltpu.sync_copy(macc.at[sl], omsg_h.at[c, sl])

        @pl.when(s == 0)
        def _():
            tail = pl.ds(16 * 624, _N - 16 * 624)
            pltpu.sync_copy(macc.at[tail], omsg_h.at[c, tail])

    return k(mm, e0)


def _sc_scatter_disp(dispt, e0, ne):
    mesh = plsc.VectorSubcoreMesh(core_axis_name="c", subcore_axis_name="s")
    nchunk = ne // _CH
    iters = -(-nchunk // _NW)

    @functools.partial(
        pl.kernel,
        out_type=jax.ShapeDtypeStruct((_NW, _DROWS, _F), jnp.float32),
        mesh=mesh,
        scratch_types=[
            pltpu.VMEM((_CH,), jnp.int32),
            pltpu.VMEM((_CH,), jnp.int32),
            pltpu.VMEM((16, _CH), jnp.float32),
            pltpu.VMEM((16, _CH), jnp.float32),
            pltpu.VMEM((_DROWS, _F), jnp.float32),
            pltpu.SemaphoreType.DMA,
            pltpu.SemaphoreType.DMA,
        ],
        compiler_params=pltpu.CompilerParams(needs_layout_passes=False),
    )
    def k(dispt_h, e0_h, odisp_h, idx0a, idx0b, drowa, drowb, dacc, semA, semB):
        c = lax.axis_index("c")
        s = lax.axis_index("s")
        wid = s * _NC + c
        pairs = -(-iters // 2)

        def zdisp(rr, c2):
            for kk in range(_F // 16):
                dacc[rr, pl.ds(kk * 16, 16)] = jnp.zeros((16,), jnp.float32)
            return c2

        lax.fori_loop(0, _DROWS, zdisp, 0)
        iota = lax.iota(jnp.int32, 16)

        def issue(ci, idx0, drow, sem):
            base = ci * _CH
            pltpu.async_copy(e0_h.at[pl.ds(base, _CH)], idx0, sem)
            pltpu.async_copy(dispt_h.at[pl.ds(0, 16), pl.ds(base, _CH)], drow, sem)

        def finish(ci, idx0, drow, sem):
            base = ci * _CH
            pltpu.make_async_copy(e0_h.at[pl.ds(base, _CH)], idx0, sem).wait()
            pltpu.make_async_copy(dispt_h.at[pl.ds(0, 16), pl.ds(base, _CH)], drow, sem).wait()
            for kk in range(_CH // 16):
                cols = kk * 16 + iota
                iv = idx0[pl.ds(kk * 16, 16)] * 4
                for cc in range(3):
                    v = plsc.load_gather(drow, [jnp.full((16,), cc, jnp.int32), cols])
                    iv4 = iv + cc
                    plsc.addupdate_scatter(
                        dacc,
                        [jnp.right_shift(iv4, 7), jnp.bitwise_and(iv4, 127)], v)

        @pl.when(wid < nchunk)
        def _():
            issue(wid, idx0a, drowa, semA)

        def body(t, carry):
            ca = wid + (2 * t) * _NW
            cb = wid + (2 * t + 1) * _NW
            ca2 = wid + (2 * t + 2) * _NW

            @pl.when(cb < nchunk)
            def _():
                issue(cb, idx0b, drowb, semB)

            @pl.when(ca < nchunk)
            def _():
                finish(ca, idx0a, drowa, semA)

            @pl.when(ca2 < nchunk)
            def _():
                issue(ca2, idx0a, drowa, semA)

            @pl.when(cb < nchunk)
            def _():
                finish(cb, idx0b, drowb, semB)

            return carry

        lax.fori_loop(0, pairs, body, 0)
        pltpu.sync_copy(dacc, odisp_h.at[wid])

    return k(dispt, e0)


def _node_update(h, mps, w_ha, w_hb, b1h, W2h, b2h):
    bn = 2000
    nmp = len(mps)

    def body(*refs):
        h_ref = refs[0]
        mp_refs = refs[1:1 + nmp]
        wha_ref, whb_ref, b1_ref, w2_ref, b2_ref, out_ref = refs[1 + nmp:]
        hv = h_ref[...]
        ms = mp_refs[0][0] + mp_refs[0][1]
        for r in mp_refs[1:]:
            ms = ms + r[0] + r[1]
        u = _silu(jnp.dot(hv, wha_ref[...], preferred_element_type=jnp.float32)
                  + jnp.dot(ms, whb_ref[...], preferred_element_type=jnp.float32)
                  + b1_ref[...])
        out_ref[...] = hv + jnp.dot(u, w2_ref[...], preferred_element_type=jnp.float32) + b2_ref[...]

    return pl.pallas_call(
        body,
        grid=(_N // bn,),
        in_specs=[pl.BlockSpec((bn, _F), lambda i: (i, 0))]
        + [pl.BlockSpec((_NC, bn, _F), lambda i: (0, i, 0)) for _ in range(nmp)]
        + [
            pl.BlockSpec((_F, _F), lambda i: (0, 0)),
            pl.BlockSpec((_F, _F), lambda i: (0, 0)),
            pl.BlockSpec((1, _F), lambda i: (0, 0)),
            pl.BlockSpec((_F, _F), lambda i: (0, 0)),
            pl.BlockSpec((1, _F), lambda i: (0, 0)),
        ],
        out_specs=pl.BlockSpec((bn, _F), lambda i: (i, 0)),
        out_shape=jax.ShapeDtypeStruct((_N, _F), jnp.float32),
    )(h, *mps, w_ha, w_hb, b1h.reshape(1, _F), W2h, b2h.reshape(1, _F))


def kernel(h, x, edges, means, stds, W1m, b1m, W2m, b2m, Wa, ba, W1x, b1x, W2x, W1h, b1h, W2h, b2h):
    e0 = edges[0]
    e1 = edges[1]
    px4 = jnp.pad(x, ((0, 0), (0, 1))).reshape(-1)
    means2 = means.reshape(1, _DF)
    coef = (-0.5 / (stds * stds)).reshape(1, _DF)

    g0, g1 = _node_pre(h, W1m[:_F], W1m[_F:2 * _F], b1m)

    mps = []
    dps = []
    lo = 0
    for ns in _SLICES:
        e0s = lax.slice_in_dim(e0, lo, lo + ns)
        e1s = lax.slice_in_dim(e1, lo, lo + ns)
        lo += ns
        pre, dvt = _sc_gather(g0, g1, px4, e0s, e1s, ns)
        mm, dispt = _edge_mlp(pre, dvt, W1m[2 * _F:], means2, coef, W2m, b2m,
                              Wa, ba, W1x, b1x, W2x, ns)
        mps.append(_sc_scatter_msg(mm, e0s, ns))
        dps.append(_sc_scatter_disp(dispt, e0s, ns))

    h_out = _node_update(h, mps, W1h[:_F], W1h[_F:], b1h, W2h, b2h)
    dpsum = dps[0]
    for d in dps[1:]:
        dpsum = dpsum + d
    x_out = x + jnp.sum(dpsum, axis=0).reshape(-1)[:_N * 4].reshape(_N, 4)[:, :3]
    return (h_out, x_out)


# speedup vs baseline: 1.0329x; 1.0329x over previous
import functools

import jax
import jax.numpy as jnp
from jax import lax
from jax.experimental import pallas as pl
from jax.experimental.pallas import tpu as pltpu
from jax.experimental.pallas import tpu_sc as plsc

_R_CUTOFF = 10.0
_N = 10000
_E = 320000
_F = 128
_DF = 16

_NC, _NS, _NW = 2, 16, 32
_CH = 128
_DROWS = 320
_SLICES = (102400, 108800, 108800)


def _silu(v):
    return v * jax.nn.sigmoid(v)


def _node_pre(h, w_a, w_b, b1m):
    bn = 2000

    def body(h_ref, wa_ref, wb_ref, b_ref, g0_ref, g1_ref):
        hv = h_ref[...]
        g0_ref[...] = jnp.dot(hv, wa_ref[...], preferred_element_type=jnp.float32) + b_ref[...]
        g1_ref[...] = jnp.dot(hv, wb_ref[...], preferred_element_type=jnp.float32)

    return pl.pallas_call(
        body,
        grid=(_N // bn,),
        in_specs=[
            pl.BlockSpec((bn, _F), lambda i: (i, 0)),
            pl.BlockSpec((_F, _F), lambda i: (0, 0)),
            pl.BlockSpec((_F, _F), lambda i: (0, 0)),
            pl.BlockSpec((1, _F), lambda i: (0, 0)),
        ],
        out_specs=[
            pl.BlockSpec((bn, _F), lambda i: (i, 0)),
            pl.BlockSpec((bn, _F), lambda i: (i, 0)),
        ],
        out_shape=[
            jax.ShapeDtypeStruct((_N, _F), jnp.float32),
            jax.ShapeDtypeStruct((_N, _F), jnp.float32),
        ],
    )(h, w_a, w_b, b1m.reshape(1, _F))


def _sc_gather(g0, g1, px4, e0, e1, ne):
    mesh = plsc.VectorSubcoreMesh(core_axis_name="c", subcore_axis_name="s")
    nchunk = ne // _CH
    iters = -(-nchunk // _NW)
    pairs = -(-iters // 2)

    @functools.partial(
        pl.kernel,
        out_type=(
            jax.ShapeDtypeStruct((ne, _F), jnp.float32),
            jax.ShapeDtypeStruct((16, ne), jnp.float32),
        ),
        mesh=mesh,
        scratch_types=[
            pltpu.VMEM((_CH,), jnp.int32),
            pltpu.VMEM((_CH,), jnp.int32),
            pltpu.VMEM((_CH,), jnp.int32),
            pltpu.VMEM((_CH,), jnp.int32),
            pltpu.VMEM((_CH, _F), jnp.float32),
            pltpu.VMEM((_CH, _F), jnp.float32),
            pltpu.VMEM((_CH, _F), jnp.float32),
            pltpu.VMEM((_CH, _F), jnp.float32),
            pltpu.VMEM((16, _CH), jnp.float32),
            pltpu.VMEM((16, _CH), jnp.float32),
            pltpu.VMEM((_N * 4,), jnp.float32),
            pltpu.SemaphoreType.DMA,
            pltpu.SemaphoreType.DMA,
        ],
        compiler_params=pltpu.CompilerParams(needs_layout_passes=False),
    )
    def k(g0_h, g1_h, px_h, e0_h, e1_h, pre_h, diff_h,
          idx0a, idx1a, idx0b, idx1b, r0a, r1a, r0b, r1b, p0a, p0b, pxv, semA, semB):
        c = lax.axis_index("c")
        s = lax.axis_index("s")
        wid = s * _NC + c
        pltpu.sync_copy(px_h, pxv)

        def zrow(rr, c2):
            for kk in range(_CH // 16):
                p0a[rr, pl.ds(kk * 16, 16)] = jnp.zeros((16,), jnp.float32)
                p0b[rr, pl.ds(kk * 16, 16)] = jnp.zeros((16,), jnp.float32)
            return c2

        lax.fori_loop(0, 16, zrow, 0)
        iota = lax.iota(jnp.int32, 16)

        def issue(ci, idx0, idx1, r0, r1, sem):
            base = ci * _CH
            pltpu.sync_copy(e0_h.at[pl.ds(base, _CH)], idx0)
            pltpu.sync_copy(e1_h.at[pl.ds(base, _CH)], idx1)
            pltpu.async_copy(g0_h.at[idx0], r0, sem)
            pltpu.async_copy(g1_h.at[idx1], r1, sem)

        def finish(ci, idx0, idx1, r0, r1, p0, sem):
            base = ci * _CH
            pltpu.make_async_copy(g0_h.at[idx0], r0, sem).wait()
            pltpu.make_async_copy(g1_h.at[idx1], r1, sem).wait()
            for kk in range(_CH // 16):
                cols = kk * 16 + iota
                iv0 = idx0[pl.ds(kk * 16, 16)] * 4
                iv1 = idx1[pl.ds(kk * 16, 16)] * 4
                for cc in range(3):
                    a = plsc.load_gather(pxv, [iv0 + cc])
                    b = plsc.load_gather(pxv, [iv1 + cc])
                    plsc.store_scatter(
                        p0, [jnp.full((16,), cc, jnp.int32), cols], a - b)

            def row(rr, c2):
                for kk in range(_F // 16):
                    sl = pl.ds(kk * 16, 16)
                    r0[rr, sl] = r0[rr, sl] + r1[rr, sl]
                return c2

            lax.fori_loop(0, _CH, row, 0)
            pltpu.sync_copy(r0, pre_h.at[pl.ds(base, _CH)])
            pltpu.sync_copy(p0, diff_h.at[pl.ds(0, 16), pl.ds(base, _CH)])

        @pl.when(wid < nchunk)
        def _():
            issue(wid, idx0a, idx1a, r0a, r1a, semA)

        def body(t, carry):
            ca = wid + (2 * t) * _NW
            cb = wid + (2 * t + 1) * _NW
            ca2 = wid + (2 * t + 2) * _NW

            @pl.when(cb < nchunk)
            def _():
                issue(cb, idx0b, idx1b, r0b, r1b, semB)

            @pl.when(ca < nchunk)
            def _():
                finish(ca, idx0a, idx1a, r0a, r1a, p0a, semA)

            @pl.when(ca2 < nchunk)
            def _():
                issue(ca2, idx0a, idx1a, r0a, r1a, semA)

            @pl.when(cb < nchunk)
            def _():
                finish(cb, idx0b, idx1b, r0b, r1b, p0b, semB)

            return carry

        lax.fori_loop(0, pairs, body, 0)

    return k(g0, g1, px4, e0, e1)


def _edge_mlp(pre, dvt, w_emb, means2, coef, W2m, b2m, Wa, ba, W1x, b1x, W2x, ne):
    be = 3200

    def body(pre_ref, dvt_ref, wemb_ref, means_ref, coef_ref, w2m_ref, b2m_ref,
             wa_ref, ba_ref, w1x_ref, b1x_ref, w2x_ref, mm_ref, dispt_ref):
        dvt = dvt_ref[...]
        dvt2 = dvt * dvt
        onescol = jnp.ones((16, 1), jnp.float32)
        d2c = lax.dot_general(dvt2, onescol, (((0,), (0,)), ((), ())))
        dist = jnp.sqrt(d2c)
        mask = (dist <= _R_CUTOFF).astype(jnp.float32)
        emb = jnp.exp(coef_ref[...] * (dist - means_ref[...]) ** 2)
        preact = pre_ref[...] + jnp.dot(emb, wemb_ref[...], preferred_element_type=jnp.float32)
        m1 = _silu(preact)
        m2 = _silu(jnp.dot(m1, w2m_ref[...], preferred_element_type=jnp.float32) + b2m_ref[...])
        att = jax.nn.sigmoid(jnp.dot(m2, wa_ref[...], preferred_element_type=jnp.float32) + ba_ref[...])
        m = m2 * att
        mm_ref[...] = m * mask
        xh = _silu(jnp.dot(m, w1x_ref[...], preferred_element_type=jnp.float32) + b1x_ref[...])
        dmt = jnp.tanh(lax.dot_general(w2x_ref[...], xh, (((0,), (1,)), ((), ()))))
        d2r = lax.dot_general(onescol, dvt2, (((0,), (0,)), ((), ())))
        distr = jnp.sqrt(d2r)
        maskr = (distr <= _R_CUTOFF).astype(jnp.float32)
        dispt_ref[...] = dvt * (dmt * maskr / distr)

    return pl.pallas_call(
        body,
        grid=(ne // be,),
        in_specs=[
            pl.BlockSpec((be, _F), lambda i: (i, 0)),
            pl.BlockSpec((16, be), lambda i: (0, i)),
            pl.BlockSpec((_DF, _F), lambda i: (0, 0)),
            pl.BlockSpec((1, _DF), lambda i: (0, 0)),
            pl.BlockSpec((1, _DF), lambda i: (0, 0)),
            pl.BlockSpec((_F, _F), lambda i: (0, 0)),
            pl.BlockSpec((1, _F), lambda i: (0, 0)),
            pl.BlockSpec((_F, 1), lambda i: (0, 0)),
            pl.BlockSpec((1, 1), lambda i: (0, 0)),
            pl.BlockSpec((_F, _F), lambda i: (0, 0)),
            pl.BlockSpec((1, _F), lambda i: (0, 0)),
            pl.BlockSpec((_F, 1), lambda i: (0, 0)),
        ],
        out_specs=[
            pl.BlockSpec((be, _F), lambda i: (i, 0)),
            pl.BlockSpec((16, be), lambda i: (0, i)),
        ],
        out_shape=[
            jax.ShapeDtypeStruct((ne, _F), jnp.float32),
            jax.ShapeDtypeStruct((16, ne), jnp.float32),
        ],
    )(pre, dvt, w_emb, means2, coef, W2m, b2m.reshape(1, _F), Wa, ba.reshape(1, 1),
      W1x, b1x.reshape(1, _F), W2x)


def _sc_scatter_msg(mm, e0, ne):
    mesh = plsc.VectorSubcoreMesh(core_axis_name="c", subcore_axis_name="s")
    nchunk = ne // _CH
    iters = -(-nchunk // _NW)

    @functools.partial(
        pl.kernel,
        out_type=jax.ShapeDtypeStruct((_NC, _N, _F), jnp.float32),
        mesh=mesh,
        scratch_types=[
            pltpu.VMEM((_CH,), jnp.int32),
            pltpu.VMEM((_CH,), jnp.int32),
            pltpu.VMEM((_CH, _F), jnp.float32),
            pltpu.VMEM((_CH, _F), jnp.float32),
            pltpu.VMEM_SHARED((_N, _F), jnp.float32),
            pltpu.SemaphoreType.DMA,
            pltpu.SemaphoreType.DMA,
        ],
        compiler_params=pltpu.CompilerParams(needs_layout_passes=False),
    )
    def k(mm_h, e0_h, omsg_h, idx0a, idx0b, mrowa, mrowb, macc, semA, semB):
        c = lax.axis_index("c")
        s = lax.axis_index("s")
        wid = s * _NC + c
        pairs = -(-iters // 2)

        def zrow(rr, c2):
            for kk in range(_F // 16):
                mrowa[rr, pl.ds(kk * 16, 16)] = jnp.zeros((16,), jnp.float32)
            return c2

        lax.fori_loop(0, _CH, zrow, 0)
        base_r = s * 624
        for t in range(5):
            pltpu.sync_copy(mrowa, macc.at[pl.ds(base_r + t * _CH, _CH)])
        plsc.subcore_barrier()

        def fire(ci, idx0, mrow, sem):
            base = ci * _CH
            pltpu.sync_copy(e0_h.at[pl.ds(base, _CH)], idx0)
            pltpu.sync_copy(mm_h.at[pl.ds(base, _CH)], mrow)
            pltpu.async_copy(mrow, macc.at[idx0], sem, add=True)

        def drain(idx0, mrow, sem):
            pltpu.make_async_copy(mrow, macc.at[idx0], sem).wait()

        @pl.when(wid < nchunk)
        def _():
            fire(wid, idx0a, mrowa, semA)

        def body(t, carry):
            cb = wid + (2 * t + 1) * _NW
            cb_prev = wid + (2 * t - 1) * _NW
            ca = wid + (2 * t) * _NW
            ca2 = wid + (2 * t + 2) * _NW

            @pl.when(jnp.logical_and(t > 0, cb_prev < nchunk))
            def _():
                drain(idx0b, mrowb, semB)

            @pl.when(cb < nchunk)
            def _():
                fire(cb, idx0b, mrowb, semB)

            @pl.when(ca < nchunk)
            def _():
                drain(idx0a, mrowa, semA)

            @pl.when(ca2 < nchunk)
            def _():
                fire(ca2, idx0a, mrowa, semA)

            return carry

        lax.fori_loop(0, pairs, body, 0)

        @pl.when(wid + (2 * pairs - 1) * _NW < nchunk)
        def _():
            drain(idx0b, mrowb, semB)

        plsc.subcore_barrier()
        sl = pl.ds(base_r, 624)
        pltpu.sync_copy(macc.at[sl], omsg_h.at[c, sl])

        @pl.when(s == 0)
        def _():
            tail = pl.ds(16 * 624, _N - 16 * 624)
            pltpu.sync_copy(macc.at[tail], omsg_h.at[c, tail])

    return k(mm, e0)


def _sc_scatter_disp(dispt, e0, ne):
    mesh = plsc.VectorSubcoreMesh(core_axis_name="c", subcore_axis_name="s")
    nchunk = ne // _CH
    iters = -(-nchunk // _NW)

    @functools.partial(
        pl.kernel,
        out_type=jax.ShapeDtypeStruct((_NW, _DROWS, _F), jnp.float32),
        mesh=mesh,
        scratch_types=[
            pltpu.VMEM((_CH,), jnp.int32),
            pltpu.VMEM((_CH,), jnp.int32),
            pltpu.VMEM((16, _CH), jnp.float32),
            pltpu.VMEM((16, _CH), jnp.float32),
            pltpu.VMEM((_DROWS, _F), jnp.float32),
            pltpu.SemaphoreType.DMA,
            pltpu.SemaphoreType.DMA,
        ],
        compiler_params=pltpu.CompilerParams(needs_layout_passes=False),
    )
    def k(dispt_h, e0_h, odisp_h, idx0a, idx0b, drowa, drowb, dacc, semA, semB):
        c = lax.axis_index("c")
        s = lax.axis_index("s")
        wid = s * _NC + c
        pairs = -(-iters // 2)

        def zdisp(rr, c2):
            for kk in range(_F // 16):
                dacc[rr, pl.ds(kk * 16, 16)] = jnp.zeros((16,), jnp.float32)
            return c2

        lax.fori_loop(0, _DROWS, zdisp, 0)
        iota = lax.iota(jnp.int32, 16)

        def issue(ci, idx0, drow, sem):
            base = ci * _CH
            pltpu.async_copy(e0_h.at[pl.ds(base, _CH)], idx0, sem)
            pltpu.async_copy(dispt_h.at[pl.ds(0, 16), pl.ds(base, _CH)], drow, sem)

        def finish(ci, idx0, drow, sem):
            base = ci * _CH
            pltpu.make_async_copy(e0_h.at[pl.ds(base, _CH)], idx0, sem).wait()
            pltpu.make_async_copy(dispt_h.at[pl.ds(0, 16), pl.ds(base, _CH)], drow, sem).wait()
            for kk in range(_CH // 16):
                cols = kk * 16 + iota
                iv = idx0[pl.ds(kk * 16, 16)] * 4
                for cc in range(3):
                    v = plsc.load_gather(drow, [jnp.full((16,), cc, jnp.int32), cols])
                    iv4 = iv + cc
                    plsc.addupdate_scatter(
                        dacc,
                        [jnp.right_shift(iv4, 7), jnp.bitwise_and(iv4, 127)], v)

        @pl.when(wid < nchunk)
        def _():
            issue(wid, idx0a, drowa, semA)

        def body(t, carry):
            ca = wid + (2 * t) * _NW
            cb = wid + (2 * t + 1) * _NW
            ca2 = wid + (2 * t + 2) * _NW

            @pl.when(cb < nchunk)
            def _():
                issue(cb, idx0b, drowb, semB)

            @pl.when(ca < nchunk)
            def _():
                finish(ca, idx0a, drowa, semA)

            @pl.when(ca2 < nchunk)
            def _():
                issue(ca2, idx0a, drowa, semA)

            @pl.when(cb < nchunk)
            def _():
                finish(cb, idx0b, drowb, semB)

            return carry

        lax.fori_loop(0, pairs, body, 0)
        pltpu.sync_copy(dacc, odisp_h.at[wid])

    return k(dispt, e0)


def _node_update(h, mps, w_ha, w_hb, b1h, W2h, b2h):
    bn = 2000
    nmp = len(mps)

    def body(*refs):
        h_ref = refs[0]
        mp_refs = refs[1:1 + nmp]
        wha_ref, whb_ref, b1_ref, w2_ref, b2_ref, out_ref = refs[1 + nmp:]
        hv = h_ref[...]
        ms = mp_refs[0][0] + mp_refs[0][1]
        for r in mp_refs[1:]:
            ms = ms + r[0] + r[1]
        u = _silu(jnp.dot(hv, wha_ref[...], preferred_element_type=jnp.float32)
                  + jnp.dot(ms, whb_ref[...], preferred_element_type=jnp.float32)
                  + b1_ref[...])
        out_ref[...] = hv + jnp.dot(u, w2_ref[...], preferred_element_type=jnp.float32) + b2_ref[...]

    return pl.pallas_call(
        body,
        grid=(_N // bn,),
        in_specs=[pl.BlockSpec((bn, _F), lambda i: (i, 0))]
        + [pl.BlockSpec((_NC, bn, _F), lambda i: (0, i, 0)) for _ in range(nmp)]
        + [
            pl.BlockSpec((_F, _F), lambda i: (0, 0)),
            pl.BlockSpec((_F, _F), lambda i: (0, 0)),
            pl.BlockSpec((1, _F), lambda i: (0, 0)),
            pl.BlockSpec((_F, _F), lambda i: (0, 0)),
            pl.BlockSpec((1, _F), lambda i: (0, 0)),
        ],
        out_specs=pl.BlockSpec((bn, _F), lambda i: (i, 0)),
        out_shape=jax.ShapeDtypeStruct((_N, _F), jnp.float32),
    )(h, *mps, w_ha, w_hb, b1h.reshape(1, _F), W2h, b2h.reshape(1, _F))


def kernel(h, x, edges, means, stds, W1m, b1m, W2m, b2m, Wa, ba, W1x, b1x, W2x, W1h, b1h, W2h, b2h):
    e0 = edges[0]
    e1 = edges[1]
    px4 = jnp.pad(x, ((0, 0), (0, 1))).reshape(-1)
    means2 = means.reshape(1, _DF)
    coef = (-0.5 / (stds * stds)).reshape(1, _DF)

    g0, g1 = _node_pre(h, W1m[:_F], W1m[_F:2 * _F], b1m)

    mps = []
    dps = []
    lo = 0
    for ns in _SLICES:
        e0s = lax.slice_in_dim(e0, lo, lo + ns)
        e1s = lax.slice_in_dim(e1, lo, lo + ns)
        lo += ns
        pre, dvt = _sc_gather(g0, g1, px4, e0s, e1s, ns)
        mm, dispt = _edge_mlp(pre, dvt, W1m[2 * _F:], means2, coef, W2m, b2m,
                              Wa, ba, W1x, b1x, W2x, ns)
        mps.append(_sc_scatter_msg(mm, e0s, ns))
        dps.append(_sc_scatter_disp(dispt, e0s, ns))

    h_out = _node_update(h, mps, W1h[:_F], W1h[_F:], b1h, W2h, b2h)
    dpsum = dps[0]
    for d in dps[1:]:
        dpsum = dpsum + d
    x_out = x + jnp.sum(dpsum, axis=0).reshape(-1)[:_N * 4].reshape(_N, 4)[:, :3]
    return (h_out, x_out)
